# baseline (speedup 1.0000x reference)
import functools

import jax
import jax.numpy as jnp
from jax import lax
from jax.experimental import pallas as pl
from jax.experimental.pallas import tpu as pltpu
from jax.experimental.pallas import tpu_sc as plsc

K = 64
EXCLUDE = 2
R_ON = 8.0
R_CUT = 10.0
R0 = 4.0
DELTA = 0.2
WALL_SCALE = 10.0
MASK_D2 = 1.0e18

_RB = 256
NW = 32
LANES = 16


def _pair_energy(r):
    x = (R0 - r) / (DELTA + 1e-12)
    sp = jnp.maximum(x, 0.0) + jnp.log1p(jnp.exp(-jnp.abs(x)))
    t = jnp.clip((R_CUT - r) / (R_CUT - R_ON), 0.0, 1.0)
    sw = t * t * (3.0 - 2.0 * t)
    return (WALL_SCALE * sp) * sw


def _sc_select(xf, yf, zf, B, L):
    NR = B * L
    RPW = NR // NW
    NG = RPW // LANES

    mesh = plsc.VectorSubcoreMesh(core_axis_name="c", subcore_axis_name="s")

    @functools.partial(
        pl.kernel,
        out_type=(
            jax.ShapeDtypeStruct((NR,), jnp.int32),
            jax.ShapeDtypeStruct((NR,), jnp.int32),
        ),
        mesh=mesh,
        compiler_params=pltpu.CompilerParams(needs_layout_passes=False),
        scratch_types=[
            pltpu.VMEM((L,), jnp.float32),
            pltpu.VMEM((L,), jnp.float32),
            pltpu.VMEM((L,), jnp.float32),
            pltpu.VMEM((L * LANES,), jnp.int32),
            pltpu.VMEM((256 * LANES,), jnp.int32),
            pltpu.VMEM((RPW,), jnp.int32),
            pltpu.VMEM((RPW,), jnp.int32),
        ],
    )
    def sc_kernel(xf_hbm, yf_hbm, zf_hbm, tau_hbm, cnt_hbm, cx, cy, cz,
                  keys, hist, taub, cntb):
        wid = lax.axis_index("s") * 2 + lax.axis_index("c")
        row0 = wid * RPW
        batch = row0 // L
        pltpu.sync_copy(xf_hbm.at[pl.ds(batch * L, L)], cx)
        pltpu.sync_copy(yf_hbm.at[pl.ds(batch * L, L)], cy)
        pltpu.sync_copy(zf_hbm.at[pl.ds(batch * L, L)], cz)

        lane = lax.iota(jnp.int32, LANES)
        zero16 = jnp.zeros((LANES,), jnp.int32)
        ones_i = jnp.full((LANES,), 1, jnp.int32)
        kvec = jnp.full((LANES,), K, jnp.int32)

        def zero_hist(h, _):
            hist[pl.ds(h * LANES, LANES)] = zero16
            return 0

        def locate(ktarget):
            def scan_bin(bb, carry):
                run, bsel, below = carry
                h = hist[pl.ds(bb * LANES, LANES)]
                cum = run + h
                m = jnp.logical_and(cum >= ktarget, run < ktarget)
                bsel = jnp.where(m, bb, bsel)
                below = jnp.where(m, run, below)
                return cum, bsel, below
            return lax.fori_loop(0, 256, scan_bin,
                                 (zero16, zero16, zero16), unroll=8)

        def do_group(g, _):
            i0 = (row0 - batch * L) + g * LANES
            ivec = i0 + lane
            xi = cx[pl.ds(i0, LANES)]
            yi = cy[pl.ds(i0, LANES)]
            zi = cz[pl.ds(i0, LANES)]

            lax.fori_loop(0, 256, zero_hist, 0, unroll=8)

            def colA(cblk, _):
                j0 = cblk * LANES
                xjv = cx[pl.ds(j0, LANES)]
                yjv = cy[pl.ds(j0, LANES)]
                zjv = cz[pl.ds(j0, LANES)]
                for l in range(LANES):
                    xj = jnp.full((LANES,), xjv[l], jnp.float32)
                    yj = jnp.full((LANES,), yjv[l], jnp.float32)
                    zj = jnp.full((LANES,), zjv[l], jnp.float32)
                    dx = xi - xj
                    dy = yi - yj
                    dz = zi - zj
                    d2 = dx * dx + dy * dy + dz * dz
                    excl = jnp.abs(ivec - (j0 + l)) <= EXCLUDE
                    d2m = jnp.where(excl, MASK_D2, d2)
                    key = lax.shift_right_logical(
                        lax.bitcast_convert_type(d2m, jnp.int32), 15)
                    keys[pl.ds((j0 + l) * LANES, LANES)] = key
                    dig = lax.shift_right_logical(key, 8)
                    plsc.addupdate_scatter(
                        hist, [dig * LANES + lane], ones_i)
                return 0
            lax.fori_loop(0, L // LANES, colA, 0)

            _, b1, below1 = locate(kvec)
            krem = kvec - below1

            lax.fori_loop(0, 256, zero_hist, 0, unroll=8)

            def colB(j, _):
                key = keys[pl.ds(j * LANES, LANES)]
                hi = lax.shift_right_logical(key, 8)
                lo = jnp.bitwise_and(key, 255)
                inb = (hi == b1).astype(jnp.int32)
                plsc.addupdate_scatter(hist, [lo * LANES + lane], inb)
                return 0
            lax.fori_loop(0, L, colB, 0, unroll=4)

            _, b2, below2 = locate(krem)

            taub[pl.ds(g * LANES, LANES)] = b1 * 256 + b2
            cntb[pl.ds(g * LANES, LANES)] = below1 + below2
            return 0

        lax.fori_loop(0, NG, do_group, 0)
        pltpu.sync_copy(taub, tau_hbm.at[pl.ds(row0, RPW)])
        pltpu.sync_copy(cntb, cnt_hbm.at[pl.ds(row0, RPW)])

    return sc_kernel(xf, yf, zf)


def _tc_body(r_ref, rt_ref, tk_ref, c_ref, o_ref):
    rows = r_ref[0]
    cols = rt_ref[0]
    tk = tk_ref[0]
    c = c_ref[0].astype(jnp.float32)
    blk = pl.program_id(1)
    RB = rows.shape[0]
    L = cols.shape[1]

    xi = rows[:, 0:1]
    yi = rows[:, 1:2]
    zi = rows[:, 2:3]
    xj = cols[0:1, :]
    yj = cols[1:2, :]
    zj = cols[2:3, :]
    dx = xi - xj
    dy = yi - yj
    dz = zi - zj
    d2 = dx * dx + dy * dy + dz * dz

    i_idx = blk * RB + lax.broadcasted_iota(jnp.int32, (RB, 1), 0)
    j_idx = lax.broadcasted_iota(jnp.int32, (1, L), 1)
    excl = jnp.abs(i_idx - j_idx) <= EXCLUDE
    d2m = jnp.where(excl, MASK_D2, d2)

    v = lax.shift_right_logical(lax.bitcast_convert_type(d2m, jnp.int32), 15)
    sel = v < tk

    r = jnp.sqrt(d2m + 1e-12)
    f = _pair_energy(r)
    s = jnp.sum(jnp.where(sel, f, 0.0), axis=1, keepdims=True)

    tau2 = lax.bitcast_convert_type(
        lax.shift_left(tk, 15) + jnp.int32(0x4000), jnp.float32)
    f_tau = _pair_energy(jnp.sqrt(tau2 + 1e-12))
    row_total = s + (K - c) * f_tau
    o_ref[0, 0] = jnp.full((8, 128), jnp.sum(row_total), jnp.float32)


@jax.jit
def kernel(R):
    B, L, _ = R.shape
    RT = jnp.swapaxes(R, 1, 2)
    xf = RT[:, 0, :].reshape(-1)
    yf = RT[:, 1, :].reshape(-1)
    zf = RT[:, 2, :].reshape(-1)
    tau_key, below = _sc_select(xf, yf, zf, B, L)
    tk3 = tau_key.reshape(B, L, 1)
    c3 = below.reshape(B, L, 1)
    nblk = L // _RB
    out = pl.pallas_call(
        _tc_body,
        grid=(B, nblk),
        in_specs=[
            pl.BlockSpec((1, _RB, 3), lambda b, k: (b, k, 0)),
            pl.BlockSpec((1, 3, L), lambda b, k: (b, 0, 0)),
            pl.BlockSpec((1, _RB, 1), lambda b, k: (b, k, 0)),
            pl.BlockSpec((1, _RB, 1), lambda b, k: (b, k, 0)),
        ],
        out_specs=pl.BlockSpec((1, 1, 8, 128), lambda b, k: (b, k, 0, 0)),
        out_shape=jax.ShapeDtypeStruct((B, nblk, 8, 128), jnp.float32),
    )(R, RT, tk3, c3)
    return jnp.sum(out[:, :, 0, 0], axis=1)


# speedup vs baseline: 1.1465x; 1.1465x over previous
import functools

import jax
import jax.numpy as jnp
from jax import lax
from jax.experimental import pallas as pl
from jax.experimental.pallas import tpu as pltpu
from jax.experimental.pallas import tpu_sc as plsc

K = 64
EXCLUDE = 2
R_ON = 8.0
R_CUT = 10.0
R0 = 4.0
DELTA = 0.2
WALL_SCALE = 10.0
MASK_D2 = 1.0e18

_RB = 256
NW = 32
LANES = 16


def _pair_energy(r):
    x = (R0 - r) / (DELTA + 1e-12)
    sp = jnp.maximum(x, 0.0) + jnp.log1p(jnp.exp(-jnp.abs(x)))
    t = jnp.clip((R_CUT - r) / (R_CUT - R_ON), 0.0, 1.0)
    sw = t * t * (3.0 - 2.0 * t)
    return (WALL_SCALE * sp) * sw


def _sc_select(xf, yf, zf, L):
    NR = L
    RPW = NR // NW
    NG = RPW // LANES

    mesh = plsc.VectorSubcoreMesh(core_axis_name="c", subcore_axis_name="s")

    @functools.partial(
        pl.kernel,
        out_type=(
            jax.ShapeDtypeStruct((NR,), jnp.int32),
            jax.ShapeDtypeStruct((NR,), jnp.int32),
        ),
        mesh=mesh,
        compiler_params=pltpu.CompilerParams(needs_layout_passes=False),
        scratch_types=[
            pltpu.VMEM((L,), jnp.float32),
            pltpu.VMEM((L,), jnp.float32),
            pltpu.VMEM((L,), jnp.float32),
            pltpu.VMEM((L * LANES,), jnp.int32),
            pltpu.VMEM((256 * LANES,), jnp.int32),
            pltpu.VMEM((RPW,), jnp.int32),
            pltpu.VMEM((RPW,), jnp.int32),
        ],
    )
    def sc_kernel(xf_hbm, yf_hbm, zf_hbm, tau_hbm, cnt_hbm, cx, cy, cz,
                  keys, hist, taub, cntb):
        wid = lax.axis_index("s") * 2 + lax.axis_index("c")
        row0 = wid * RPW
        pltpu.sync_copy(xf_hbm, cx)
        pltpu.sync_copy(yf_hbm, cy)
        pltpu.sync_copy(zf_hbm, cz)

        lane = lax.iota(jnp.int32, LANES)
        zero16 = jnp.zeros((LANES,), jnp.int32)
        ones_i = jnp.full((LANES,), 1, jnp.int32)
        kvec = jnp.full((LANES,), K, jnp.int32)

        def zero_hist(h, _):
            hist[pl.ds(h * LANES, LANES)] = zero16
            return 0

        def locate(ktarget):
            def scan_bin(bb, carry):
                run, bsel, below = carry
                h = hist[pl.ds(bb * LANES, LANES)]
                cum = run + h
                m = jnp.logical_and(cum >= ktarget, run < ktarget)
                bsel = jnp.where(m, bb, bsel)
                below = jnp.where(m, run, below)
                return cum, bsel, below
            return lax.fori_loop(0, 256, scan_bin,
                                 (zero16, zero16, zero16), unroll=8)

        def do_group(g, _):
            i0 = row0 + g * LANES
            ivec = i0 + lane
            xi = cx[pl.ds(i0, LANES)]
            yi = cy[pl.ds(i0, LANES)]
            zi = cz[pl.ds(i0, LANES)]

            lax.fori_loop(0, 256, zero_hist, 0, unroll=8)

            def colA(cblk, _):
                j0 = cblk * LANES
                xjv = cx[pl.ds(j0, LANES)]
                yjv = cy[pl.ds(j0, LANES)]
                zjv = cz[pl.ds(j0, LANES)]
                for l in range(LANES):
                    xj = jnp.full((LANES,), xjv[l], jnp.float32)
                    yj = jnp.full((LANES,), yjv[l], jnp.float32)
                    zj = jnp.full((LANES,), zjv[l], jnp.float32)
                    dx = xi - xj
                    dy = yi - yj
                    dz = zi - zj
                    d2 = dx * dx + dy * dy + dz * dz
                    excl = jnp.abs(ivec - (j0 + l)) <= EXCLUDE
                    d2m = jnp.where(excl, MASK_D2, d2)
                    key = lax.shift_right_logical(
                        lax.bitcast_convert_type(d2m, jnp.int32), 15)
                    keys[pl.ds((j0 + l) * LANES, LANES)] = key
                    dig = lax.shift_right_logical(key, 8)
                    plsc.addupdate_scatter(
                        hist, [dig * LANES + lane], ones_i)
                return 0
            lax.fori_loop(0, L // LANES, colA, 0)

            _, b1, below1 = locate(kvec)
            krem = kvec - below1

            lax.fori_loop(0, 256, zero_hist, 0, unroll=8)

            def colB(j, _):
                key = keys[pl.ds(j * LANES, LANES)]
                hi = lax.shift_right_logical(key, 8)
                lo = jnp.bitwise_and(key, 255)
                inb = (hi == b1).astype(jnp.int32)
                plsc.addupdate_scatter(hist, [lo * LANES + lane], inb)
                return 0
            lax.fori_loop(0, L, colB, 0, unroll=4)

            _, b2, below2 = locate(krem)

            taub[pl.ds(g * LANES, LANES)] = b1 * 256 + b2
            cntb[pl.ds(g * LANES, LANES)] = below1 + below2
            return 0

        lax.fori_loop(0, NG, do_group, 0)
        pltpu.sync_copy(taub, tau_hbm.at[pl.ds(row0, RPW)])
        pltpu.sync_copy(cntb, cnt_hbm.at[pl.ds(row0, RPW)])

    return sc_kernel(xf, yf, zf)


def _tc_body(r_ref, rt_ref, tk_ref, c_ref, o_ref):
    rows = r_ref[0]
    cols = rt_ref[0]
    tk = tk_ref[0]
    c = c_ref[0].astype(jnp.float32)
    blk = pl.program_id(1)
    RB = rows.shape[0]
    L = cols.shape[1]

    xi = rows[:, 0:1]
    yi = rows[:, 1:2]
    zi = rows[:, 2:3]
    xj = cols[0:1, :]
    yj = cols[1:2, :]
    zj = cols[2:3, :]
    dx = xi - xj
    dy = yi - yj
    dz = zi - zj
    d2 = dx * dx + dy * dy + dz * dz

    i_idx = blk * RB + lax.broadcasted_iota(jnp.int32, (RB, 1), 0)
    j_idx = lax.broadcasted_iota(jnp.int32, (1, L), 1)
    excl = jnp.abs(i_idx - j_idx) <= EXCLUDE
    d2m = jnp.where(excl, MASK_D2, d2)

    v = lax.shift_right_logical(lax.bitcast_convert_type(d2m, jnp.int32), 15)
    sel = v < tk

    r = jnp.sqrt(d2m + 1e-12)
    f = _pair_energy(r)
    s = jnp.sum(jnp.where(sel, f, 0.0), axis=1, keepdims=True)

    tau2 = lax.bitcast_convert_type(
        lax.shift_left(tk, 15) + jnp.int32(0x4000), jnp.float32)
    f_tau = _pair_energy(jnp.sqrt(tau2 + 1e-12))
    row_total = s + (K - c) * f_tau
    o_ref[0, 0] = jnp.full((8, 128), jnp.sum(row_total), jnp.float32)


@jax.jit
def kernel(R):
    B, L, _ = R.shape
    RT = jnp.swapaxes(R, 1, 2)
    nblk = L // _RB
    totals = []
    for b in range(B):
        xf = RT[b, 0, :]
        yf = RT[b, 1, :]
        zf = RT[b, 2, :]
        tau_key, below = _sc_select(xf, yf, zf, L)
        tk3 = tau_key.reshape(1, L, 1)
        c3 = below.reshape(1, L, 1)
        out = pl.pallas_call(
            _tc_body,
            grid=(1, nblk),
            in_specs=[
                pl.BlockSpec((1, _RB, 3), lambda i, k: (i, k, 0)),
                pl.BlockSpec((1, 3, L), lambda i, k: (i, 0, 0)),
                pl.BlockSpec((1, _RB, 1), lambda i, k: (i, k, 0)),
                pl.BlockSpec((1, _RB, 1), lambda i, k: (i, k, 0)),
            ],
            out_specs=pl.BlockSpec((1, 1, 8, 128),
                                   lambda i, k: (i, k, 0, 0)),
            out_shape=jax.ShapeDtypeStruct((1, nblk, 8, 128), jnp.float32),
        )(R[b:b + 1], RT[b:b + 1], tk3, c3)
        totals.append(jnp.sum(out[0, :, 0, 0]))
    return jnp.stack(totals)


# speedup vs baseline: 1.1866x; 1.0350x over previous
import functools

import jax
import jax.numpy as jnp
from jax import lax
from jax.experimental import pallas as pl
from jax.experimental.pallas import tpu as pltpu
from jax.experimental.pallas import tpu_sc as plsc

K = 64
EXCLUDE = 2
R_ON = 8.0
R_CUT = 10.0
R0 = 4.0
DELTA = 0.2
WALL_SCALE = 10.0
MASK_D2 = 1.0e18

_RB = 256
NW = 32
LANES = 16


def _pair_energy(r):
    x = (R0 - r) / (DELTA + 1e-12)
    sp = jnp.maximum(x, 0.0) + jnp.log1p(jnp.exp(-jnp.abs(x)))
    t = jnp.clip((R_CUT - r) / (R_CUT - R_ON), 0.0, 1.0)
    sw = t * t * (3.0 - 2.0 * t)
    return (WALL_SCALE * sp) * sw


def _sc_select(xf, yf, zf, L):
    NR = L
    RPW = NR // NW
    NG = RPW // LANES

    mesh = plsc.VectorSubcoreMesh(core_axis_name="c", subcore_axis_name="s")

    @functools.partial(
        pl.kernel,
        out_type=(
            jax.ShapeDtypeStruct((NR,), jnp.int32),
            jax.ShapeDtypeStruct((NR,), jnp.int32),
        ),
        mesh=mesh,
        compiler_params=pltpu.CompilerParams(needs_layout_passes=False),
        scratch_types=[
            pltpu.VMEM((L + 32,), jnp.float32),
            pltpu.VMEM((L + 32,), jnp.float32),
            pltpu.VMEM((L + 32,), jnp.float32),
            pltpu.VMEM((L * LANES,), jnp.int32),
            pltpu.VMEM((256 * LANES,), jnp.int32),
            pltpu.VMEM((RPW,), jnp.int32),
            pltpu.VMEM((RPW,), jnp.int32),
        ],
    )
    def sc_kernel(xf_hbm, yf_hbm, zf_hbm, tau_hbm, cnt_hbm, cx, cy, cz,
                  keys, hist, taub, cntb):
        wid = lax.axis_index("s") * 2 + lax.axis_index("c")
        row0 = wid * RPW
        pltpu.sync_copy(xf_hbm, cx.at[pl.ds(16, L)])
        pltpu.sync_copy(yf_hbm, cy.at[pl.ds(16, L)])
        pltpu.sync_copy(zf_hbm, cz.at[pl.ds(16, L)])

        lane = lax.iota(jnp.int32, LANES)
        zero16 = jnp.zeros((LANES,), jnp.int32)
        ones_i = jnp.full((LANES,), 1, jnp.int32)
        neg_i = jnp.full((LANES,), -1, jnp.int32)
        kvec = jnp.full((LANES,), K, jnp.int32)

        def zero_hist(h, _):
            hist[pl.ds(h * LANES, LANES)] = zero16
            return 0

        def locate(ktarget):
            def scan_bin(bb, carry):
                run, bsel, below = carry
                h = hist[pl.ds(bb * LANES, LANES)]
                cum = run + h
                m = jnp.logical_and(cum >= ktarget, run < ktarget)
                bsel = jnp.where(m, bb, bsel)
                below = jnp.where(m, run, below)
                return cum, bsel, below
            return lax.fori_loop(0, 256, scan_bin,
                                 (zero16, zero16, zero16), unroll=8)

        def do_group(g, _):
            i0 = row0 + g * LANES
            ivec = i0 + lane
            xi = cx[pl.ds(16 + i0, LANES)]
            yi = cy[pl.ds(16 + i0, LANES)]
            zi = cz[pl.ds(16 + i0, LANES)]

            lax.fori_loop(0, 256, zero_hist, 0, unroll=8)

            def colA(cblk, _):
                j0 = cblk * LANES
                xjv = cx[pl.ds(16 + j0, LANES)]
                yjv = cy[pl.ds(16 + j0, LANES)]
                zjv = cz[pl.ds(16 + j0, LANES)]
                for l in range(LANES):
                    xj = jnp.full((LANES,), xjv[l], jnp.float32)
                    yj = jnp.full((LANES,), yjv[l], jnp.float32)
                    zj = jnp.full((LANES,), zjv[l], jnp.float32)
                    dx = xi - xj
                    dy = yi - yj
                    dz = zi - zj
                    d2 = dx * dx + dy * dy + dz * dz
                    key = lax.shift_right_logical(
                        lax.bitcast_convert_type(d2, jnp.int32), 15)
                    keys[pl.ds((j0 + l) * LANES, LANES)] = key
                    dig = lax.shift_right_logical(key, 8)
                    plsc.addupdate_scatter(
                        hist, [dig * LANES + lane], ones_i)
                return 0
            lax.fori_loop(0, L // LANES, colA, 0)

            def excl_keys():
                out = []
                for o in range(-EXCLUDE, EXCLUDE + 1):
                    xo = cx[pl.ds(16 + i0 + o, LANES)]
                    yo = cy[pl.ds(16 + i0 + o, LANES)]
                    zo = cz[pl.ds(16 + i0 + o, LANES)]
                    dx = xi - xo
                    dy = yi - yo
                    dz = zi - zo
                    d2 = dx * dx + dy * dy + dz * dz
                    key = lax.shift_right_logical(
                        lax.bitcast_convert_type(d2, jnp.int32), 15)
                    jv = ivec + o
                    valid = jnp.logical_and(jv >= 0, jv < L)
                    out.append((key, valid))
                return out

            exk = excl_keys()
            for key, valid in exk:
                dig = lax.shift_right_logical(key, 8)
                plsc.addupdate_scatter(
                    hist, [dig * LANES + lane], neg_i, mask=valid)

            _, b1, below1 = locate(kvec)
            krem = kvec - below1

            lax.fori_loop(0, 256, zero_hist, 0, unroll=8)

            def colB(j, _):
                key = keys[pl.ds(j * LANES, LANES)]
                hi = lax.shift_right_logical(key, 8)
                lo = jnp.bitwise_and(key, 255)
                plsc.addupdate_scatter(hist, [lo * LANES + lane], ones_i,
                                       mask=hi == b1)
                return 0
            lax.fori_loop(0, L, colB, 0, unroll=4)

            for key, valid in exk:
                hi = lax.shift_right_logical(key, 8)
                lo = jnp.bitwise_and(key, 255)
                m2 = jnp.logical_and(valid, hi == b1)
                plsc.addupdate_scatter(hist, [lo * LANES + lane], neg_i,
                                       mask=m2)

            _, b2, below2 = locate(krem)

            taub[pl.ds(g * LANES, LANES)] = b1 * 256 + b2
            cntb[pl.ds(g * LANES, LANES)] = below1 + below2
            return 0

        lax.fori_loop(0, NG, do_group, 0)
        pltpu.sync_copy(taub, tau_hbm.at[pl.ds(row0, RPW)])
        pltpu.sync_copy(cntb, cnt_hbm.at[pl.ds(row0, RPW)])

    return sc_kernel(xf, yf, zf)


def _tc_body(r_ref, rt_ref, tk_ref, c_ref, o_ref):
    rows = r_ref[0]
    cols = rt_ref[0]
    tk = tk_ref[0]
    c = c_ref[0].astype(jnp.float32)
    blk = pl.program_id(1)
    RB = rows.shape[0]
    L = cols.shape[1]

    xi = rows[:, 0:1]
    yi = rows[:, 1:2]
    zi = rows[:, 2:3]
    xj = cols[0:1, :]
    yj = cols[1:2, :]
    zj = cols[2:3, :]
    dx = xi - xj
    dy = yi - yj
    dz = zi - zj
    d2 = dx * dx + dy * dy + dz * dz

    i_idx = blk * RB + lax.broadcasted_iota(jnp.int32, (RB, 1), 0)
    j_idx = lax.broadcasted_iota(jnp.int32, (1, L), 1)
    excl = jnp.abs(i_idx - j_idx) <= EXCLUDE
    d2m = jnp.where(excl, MASK_D2, d2)

    v = lax.shift_right_logical(lax.bitcast_convert_type(d2m, jnp.int32), 15)
    sel = v < tk

    r = jnp.sqrt(d2m + 1e-12)
    f = _pair_energy(r)
    s = jnp.sum(jnp.where(sel, f, 0.0), axis=1, keepdims=True)

    tau2 = lax.bitcast_convert_type(
        lax.shift_left(tk, 15) + jnp.int32(0x4000), jnp.float32)
    f_tau = _pair_energy(jnp.sqrt(tau2 + 1e-12))
    row_total = s + (K - c) * f_tau
    o_ref[0, 0] = jnp.full((8, 128), jnp.sum(row_total), jnp.float32)


@jax.jit
def kernel(R):
    B, L, _ = R.shape
    RT = jnp.swapaxes(R, 1, 2)
    nblk = L // _RB
    totals = []
    for b in range(B):
        xf = RT[b, 0, :]
        yf = RT[b, 1, :]
        zf = RT[b, 2, :]
        tau_key, below = _sc_select(xf, yf, zf, L)
        tk3 = tau_key.reshape(1, L, 1)
        c3 = below.reshape(1, L, 1)
        out = pl.pallas_call(
            _tc_body,
            grid=(1, nblk),
            in_specs=[
                pl.BlockSpec((1, _RB, 3), lambda i, k: (i, k, 0)),
                pl.BlockSpec((1, 3, L), lambda i, k: (i, 0, 0)),
                pl.BlockSpec((1, _RB, 1), lambda i, k: (i, k, 0)),
                pl.BlockSpec((1, _RB, 1), lambda i, k: (i, k, 0)),
            ],
            out_specs=pl.BlockSpec((1, 1, 8, 128),
                                   lambda i, k: (i, k, 0, 0)),
            out_shape=jax.ShapeDtypeStruct((1, nblk, 8, 128), jnp.float32),
        )(R[b:b + 1], RT[b:b + 1], tk3, c3)
        totals.append(jnp.sum(out[0, :, 0, 0]))
    return jnp.stack(totals)


# speedup vs baseline: 1.7469x; 1.4721x over previous
import functools

import jax
import jax.numpy as jnp
from jax import lax
from jax.experimental import pallas as pl
from jax.experimental.pallas import tpu as pltpu
from jax.experimental.pallas import tpu_sc as plsc

K = 64
EXCLUDE = 2
R_ON = 8.0
R_CUT = 10.0
R0 = 4.0
DELTA = 0.2
WALL_SCALE = 10.0
MASK_D2 = 1.0e18

_RB = 256
NW = 32
LANES = 16
SPLIT = 1024


def _pair_energy(r):
    x = (R0 - r) / (DELTA + 1e-12)
    sp = jnp.maximum(x, 0.0) + jnp.log1p(jnp.exp(-jnp.abs(x)))
    t = jnp.clip((R_CUT - r) / (R_CUT - R_ON), 0.0, 1.0)
    sw = t * t * (3.0 - 2.0 * t)
    return (WALL_SCALE * sp) * sw


def _sc_select(xf, yf, zf, L):
    NR = L - SPLIT
    RPW = NR // NW
    NG = RPW // LANES

    mesh = plsc.VectorSubcoreMesh(core_axis_name="c", subcore_axis_name="s")

    @functools.partial(
        pl.kernel,
        out_type=(
            jax.ShapeDtypeStruct((NR,), jnp.int32),
            jax.ShapeDtypeStruct((NR,), jnp.int32),
        ),
        mesh=mesh,
        compiler_params=pltpu.CompilerParams(needs_layout_passes=False),
        scratch_types=[
            pltpu.VMEM((L + 32,), jnp.float32),
            pltpu.VMEM((L + 32,), jnp.float32),
            pltpu.VMEM((L + 32,), jnp.float32),
            pltpu.VMEM((L * LANES,), jnp.int32),
            pltpu.VMEM((256 * LANES,), jnp.int32),
            pltpu.VMEM((RPW,), jnp.int32),
            pltpu.VMEM((RPW,), jnp.int32),
        ],
    )
    def sc_kernel(xf_hbm, yf_hbm, zf_hbm, tau_hbm, cnt_hbm, cx, cy, cz,
                  keys, hist, taub, cntb):
        wid = lax.axis_index("s") * 2 + lax.axis_index("c")
        out0 = wid * RPW
        row0 = SPLIT + out0
        pltpu.sync_copy(xf_hbm, cx.at[pl.ds(16, L)])
        pltpu.sync_copy(yf_hbm, cy.at[pl.ds(16, L)])
        pltpu.sync_copy(zf_hbm, cz.at[pl.ds(16, L)])

        lane = lax.iota(jnp.int32, LANES)
        zero16 = jnp.zeros((LANES,), jnp.int32)
        ones_i = jnp.full((LANES,), 1, jnp.int32)
        neg_i = jnp.full((LANES,), -1, jnp.int32)
        kvec = jnp.full((LANES,), K, jnp.int32)

        def zero_hist(h, _):
            hist[pl.ds(h * LANES, LANES)] = zero16
            return 0

        def locate(ktarget):
            def scan_bin(bb, carry):
                run, bsel, below = carry
                h = hist[pl.ds(bb * LANES, LANES)]
                cum = run + h
                m = jnp.logical_and(cum >= ktarget, run < ktarget)
                bsel = jnp.where(m, bb, bsel)
                below = jnp.where(m, run, below)
                return cum, bsel, below
            return lax.fori_loop(0, 256, scan_bin,
                                 (zero16, zero16, zero16), unroll=8)

        def do_group(g, _):
            i0 = row0 + g * LANES
            ivec = i0 + lane
            xi = cx[pl.ds(16 + i0, LANES)]
            yi = cy[pl.ds(16 + i0, LANES)]
            zi = cz[pl.ds(16 + i0, LANES)]

            lax.fori_loop(0, 256, zero_hist, 0, unroll=8)

            def colA(cblk, _):
                j0 = cblk * LANES
                xjv = cx[pl.ds(16 + j0, LANES)]
                yjv = cy[pl.ds(16 + j0, LANES)]
                zjv = cz[pl.ds(16 + j0, LANES)]
                for l in range(LANES):
                    xj = jnp.full((LANES,), xjv[l], jnp.float32)
                    yj = jnp.full((LANES,), yjv[l], jnp.float32)
                    zj = jnp.full((LANES,), zjv[l], jnp.float32)
                    dx = xi - xj
                    dy = yi - yj
                    dz = zi - zj
                    d2 = dx * dx + dy * dy + dz * dz
                    key = lax.shift_right_logical(
                        lax.bitcast_convert_type(d2, jnp.int32), 15)
                    keys[pl.ds((j0 + l) * LANES, LANES)] = key
                    dig = lax.shift_right_logical(key, 8)
                    plsc.addupdate_scatter(
                        hist, [dig * LANES + lane], ones_i)
                return 0
            lax.fori_loop(0, L // LANES, colA, 0)

            def excl_keys():
                out = []
                for o in range(-EXCLUDE, EXCLUDE + 1):
                    xo = cx[pl.ds(16 + i0 + o, LANES)]
                    yo = cy[pl.ds(16 + i0 + o, LANES)]
                    zo = cz[pl.ds(16 + i0 + o, LANES)]
                    dx = xi - xo
                    dy = yi - yo
                    dz = zi - zo
                    d2 = dx * dx + dy * dy + dz * dz
                    key = lax.shift_right_logical(
                        lax.bitcast_convert_type(d2, jnp.int32), 15)
                    jv = ivec + o
                    valid = jnp.logical_and(jv >= 0, jv < L)
                    out.append((key, valid))
                return out

            exk = excl_keys()
            for key, valid in exk:
                dig = lax.shift_right_logical(key, 8)
                plsc.addupdate_scatter(
                    hist, [dig * LANES + lane], neg_i, mask=valid)

            _, b1, below1 = locate(kvec)
            krem = kvec - below1

            lax.fori_loop(0, 256, zero_hist, 0, unroll=8)

            def colB(j, _):
                key = keys[pl.ds(j * LANES, LANES)]
                hi = lax.shift_right_logical(key, 8)
                lo = jnp.bitwise_and(key, 255)
                plsc.addupdate_scatter(hist, [lo * LANES + lane], ones_i,
                                       mask=hi == b1)
                return 0
            lax.fori_loop(0, L, colB, 0, unroll=4)

            for key, valid in exk:
                hi = lax.shift_right_logical(key, 8)
                lo = jnp.bitwise_and(key, 255)
                m2 = jnp.logical_and(valid, hi == b1)
                plsc.addupdate_scatter(hist, [lo * LANES + lane], neg_i,
                                       mask=m2)

            _, b2, below2 = locate(krem)

            taub[pl.ds(g * LANES, LANES)] = b1 * 256 + b2
            cntb[pl.ds(g * LANES, LANES)] = below1 + below2
            return 0

        lax.fori_loop(0, NG, do_group, 0)
        pltpu.sync_copy(taub, tau_hbm.at[pl.ds(out0, RPW)])
        pltpu.sync_copy(cntb, cnt_hbm.at[pl.ds(out0, RPW)])

    return sc_kernel(xf, yf, zf)


def _tc_bisect_body(r_ref, rt_ref, o_ref):
    rows = r_ref[0]
    cols = rt_ref[0]
    blk = pl.program_id(1)
    RB = rows.shape[0]
    L = cols.shape[1]

    xi = rows[:, 0:1]
    yi = rows[:, 1:2]
    zi = rows[:, 2:3]
    xj = cols[0:1, :]
    yj = cols[1:2, :]
    zj = cols[2:3, :]
    dx = xi - xj
    dy = yi - yj
    dz = zi - zj
    d2 = dx * dx + dy * dy + dz * dz

    i_idx = blk * RB + lax.broadcasted_iota(jnp.int32, (RB, 1), 0)
    j_idx = lax.broadcasted_iota(jnp.int32, (1, L), 1)
    excl = jnp.abs(i_idx - j_idx) <= EXCLUDE
    d2m = jnp.where(excl, MASK_D2, d2)

    v = lax.shift_right_logical(lax.bitcast_convert_type(d2m, jnp.int32), 15)

    def bit_step(i, q):
        bit = lax.shift_left(jnp.int32(1), jnp.int32(15) - i)
        cand = q | bit
        cnt = jnp.sum((v < cand).astype(jnp.int32), axis=1, keepdims=True)
        return jnp.where(cnt <= K - 1, cand, q)

    q0 = jnp.zeros((RB, 1), dtype=jnp.int32)
    q = lax.fori_loop(0, 16, bit_step, q0)

    tau2 = lax.bitcast_convert_type(
        lax.shift_left(q, 15) + jnp.int32(0x4000), jnp.float32)
    sel = v < q
    c = jnp.sum(sel.astype(jnp.float32), axis=1, keepdims=True)

    r = jnp.sqrt(d2m + 1e-12)
    f = _pair_energy(r)
    ssum = jnp.sum(jnp.where(sel, f, 0.0), axis=1, keepdims=True)
    f_tau = _pair_energy(jnp.sqrt(tau2 + 1e-12))
    row_total = ssum + (K - c) * f_tau
    o_ref[0, 0] = jnp.full((8, 128), jnp.sum(row_total), jnp.float32)


def _tc_body(row_off, r_ref, rt_ref, tk_ref, c_ref, o_ref):
    rows = r_ref[0]
    cols = rt_ref[0]
    tk = tk_ref[0]
    c = c_ref[0].astype(jnp.float32)
    blk = pl.program_id(1)
    RB = rows.shape[0]
    L = cols.shape[1]

    xi = rows[:, 0:1]
    yi = rows[:, 1:2]
    zi = rows[:, 2:3]
    xj = cols[0:1, :]
    yj = cols[1:2, :]
    zj = cols[2:3, :]
    dx = xi - xj
    dy = yi - yj
    dz = zi - zj
    d2 = dx * dx + dy * dy + dz * dz

    i_idx = row_off + blk * RB + lax.broadcasted_iota(jnp.int32, (RB, 1), 0)
    j_idx = lax.broadcasted_iota(jnp.int32, (1, L), 1)
    excl = jnp.abs(i_idx - j_idx) <= EXCLUDE
    d2m = jnp.where(excl, MASK_D2, d2)

    v = lax.shift_right_logical(lax.bitcast_convert_type(d2m, jnp.int32), 15)
    sel = v < tk

    r = jnp.sqrt(d2m + 1e-12)
    f = _pair_energy(r)
    s = jnp.sum(jnp.where(sel, f, 0.0), axis=1, keepdims=True)

    tau2 = lax.bitcast_convert_type(
        lax.shift_left(tk, 15) + jnp.int32(0x4000), jnp.float32)
    f_tau = _pair_energy(jnp.sqrt(tau2 + 1e-12))
    row_total = s + (K - c) * f_tau
    o_ref[0, 0] = jnp.full((8, 128), jnp.sum(row_total), jnp.float32)


@jax.jit
def kernel(R):
    B, L, _ = R.shape
    RT = jnp.swapaxes(R, 1, 2)
    nblk_a = SPLIT // _RB
    nblk_b = (L - SPLIT) // _RB
    totals = []
    for b in range(B):
        xf = RT[b, 0, :]
        yf = RT[b, 1, :]
        zf = RT[b, 2, :]
        tau_key, below = _sc_select(xf, yf, zf, L)
        tk3 = tau_key.reshape(1, L - SPLIT, 1)
        c3 = below.reshape(1, L - SPLIT, 1)
        out_a = pl.pallas_call(
            _tc_bisect_body,
            grid=(1, nblk_a),
            in_specs=[
                pl.BlockSpec((1, _RB, 3), lambda i, k: (i, k, 0)),
                pl.BlockSpec((1, 3, L), lambda i, k: (i, 0, 0)),
            ],
            out_specs=pl.BlockSpec((1, 1, 8, 128),
                                   lambda i, k: (i, k, 0, 0)),
            out_shape=jax.ShapeDtypeStruct((1, nblk_a, 8, 128),
                                           jnp.float32),
        )(R[b:b + 1, :SPLIT], RT[b:b + 1])
        out_b = pl.pallas_call(
            functools.partial(_tc_body, SPLIT),
            grid=(1, nblk_b),
            in_specs=[
                pl.BlockSpec((1, _RB, 3), lambda i, k: (i, k, 0)),
                pl.BlockSpec((1, 3, L), lambda i, k: (i, 0, 0)),
                pl.BlockSpec((1, _RB, 1), lambda i, k: (i, k, 0)),
                pl.BlockSpec((1, _RB, 1), lambda i, k: (i, k, 0)),
            ],
            out_specs=pl.BlockSpec((1, 1, 8, 128),
                                   lambda i, k: (i, k, 0, 0)),
            out_shape=jax.ShapeDtypeStruct((1, nblk_b, 8, 128),
                                           jnp.float32),
        )(R[b:b + 1, SPLIT:], RT[b:b + 1], tk3, c3)
        totals.append(jnp.sum(out_a[0, :, 0, 0]) +
                      jnp.sum(out_b[0, :, 0, 0]))
    return jnp.stack(totals)


# speedup vs baseline: 1.7630x; 1.0092x over previous
import functools

import jax
import jax.numpy as jnp
from jax import lax
from jax.experimental import pallas as pl
from jax.experimental.pallas import tpu as pltpu
from jax.experimental.pallas import tpu_sc as plsc

K = 64
EXCLUDE = 2
R_ON = 8.0
R_CUT = 10.0
R0 = 4.0
DELTA = 0.2
WALL_SCALE = 10.0
MASK_D2 = 1.0e18

_RB = 256
NW = 32
LANES = 16
SPLIT = 1024


def _pair_energy(r):
    x = (R0 - r) / (DELTA + 1e-12)
    sp = jnp.maximum(x, 0.0) + jnp.log1p(jnp.exp(-jnp.abs(x)))
    t = jnp.clip((R_CUT - r) / (R_CUT - R_ON), 0.0, 1.0)
    sw = t * t * (3.0 - 2.0 * t)
    return (WALL_SCALE * sp) * sw


def _sc_select(xf, yf, zf, L):
    NR = L - SPLIT
    RPW = NR // NW
    NG = RPW // LANES

    mesh = plsc.VectorSubcoreMesh(core_axis_name="c", subcore_axis_name="s")

    @functools.partial(
        pl.kernel,
        out_type=(
            jax.ShapeDtypeStruct((NR,), jnp.int32),
            jax.ShapeDtypeStruct((NR,), jnp.int32),
        ),
        mesh=mesh,
        compiler_params=pltpu.CompilerParams(needs_layout_passes=False),
        scratch_types=[
            pltpu.VMEM((L + 32,), jnp.float32),
            pltpu.VMEM((L + 32,), jnp.float32),
            pltpu.VMEM((L + 32,), jnp.float32),
            pltpu.VMEM((L * LANES,), jnp.int32),
            pltpu.VMEM((256 * LANES,), jnp.int32),
            pltpu.VMEM((RPW,), jnp.int32),
            pltpu.VMEM((RPW,), jnp.int32),
        ],
    )
    def sc_kernel(xf_hbm, yf_hbm, zf_hbm, tau_hbm, cnt_hbm, cx, cy, cz,
                  keys, hist, taub, cntb):
        wid = lax.axis_index("s") * 2 + lax.axis_index("c")
        out0 = wid * RPW
        row0 = SPLIT + out0
        pltpu.sync_copy(xf_hbm, cx.at[pl.ds(16, L)])
        pltpu.sync_copy(yf_hbm, cy.at[pl.ds(16, L)])
        pltpu.sync_copy(zf_hbm, cz.at[pl.ds(16, L)])

        lane = lax.iota(jnp.int32, LANES)
        zero16 = jnp.zeros((LANES,), jnp.int32)
        ones_i = jnp.full((LANES,), 1, jnp.int32)
        neg_i = jnp.full((LANES,), -1, jnp.int32)
        kvec = jnp.full((LANES,), K, jnp.int32)

        def zero_hist(h, _):
            hist[pl.ds(h * LANES, LANES)] = zero16
            return 0

        def locate(ktarget):
            def scan_bin(bb, carry):
                run, bsel, below = carry
                h = hist[pl.ds(bb * LANES, LANES)]
                cum = run + h
                m = jnp.logical_and(cum >= ktarget, run < ktarget)
                bsel = jnp.where(m, bb, bsel)
                below = jnp.where(m, run, below)
                return cum, bsel, below
            return lax.fori_loop(0, 256, scan_bin,
                                 (zero16, zero16, zero16), unroll=8)

        def do_group(g, _):
            i0 = row0 + g * LANES
            ivec = i0 + lane
            xi = cx[pl.ds(16 + i0, LANES)]
            yi = cy[pl.ds(16 + i0, LANES)]
            zi = cz[pl.ds(16 + i0, LANES)]

            lax.fori_loop(0, 256, zero_hist, 0, unroll=8)

            def colA(cblk, _):
                j0 = cblk * LANES
                xjv = cx[pl.ds(16 + j0, LANES)]
                yjv = cy[pl.ds(16 + j0, LANES)]
                zjv = cz[pl.ds(16 + j0, LANES)]
                for l in range(LANES):
                    xj = jnp.full((LANES,), xjv[l], jnp.float32)
                    yj = jnp.full((LANES,), yjv[l], jnp.float32)
                    zj = jnp.full((LANES,), zjv[l], jnp.float32)
                    dx = xi - xj
                    dy = yi - yj
                    dz = zi - zj
                    d2 = dx * dx + dy * dy + dz * dz
                    key = lax.shift_right_logical(
                        lax.bitcast_convert_type(d2, jnp.int32), 15)
                    keys[pl.ds((j0 + l) * LANES, LANES)] = key
                    dig = lax.shift_right_logical(key, 8)
                    plsc.addupdate_scatter(
                        hist, [dig * LANES + lane], ones_i)
                return 0
            lax.fori_loop(0, L // LANES, colA, 0)

            def excl_keys():
                out = []
                for o in range(-EXCLUDE, EXCLUDE + 1):
                    xo = cx[pl.ds(16 + i0 + o, LANES)]
                    yo = cy[pl.ds(16 + i0 + o, LANES)]
                    zo = cz[pl.ds(16 + i0 + o, LANES)]
                    dx = xi - xo
                    dy = yi - yo
                    dz = zi - zo
                    d2 = dx * dx + dy * dy + dz * dz
                    key = lax.shift_right_logical(
                        lax.bitcast_convert_type(d2, jnp.int32), 15)
                    jv = ivec + o
                    valid = jnp.logical_and(jv >= 0, jv < L)
                    out.append((key, valid))
                return out

            exk = excl_keys()
            for key, valid in exk:
                dig = lax.shift_right_logical(key, 8)
                plsc.addupdate_scatter(
                    hist, [dig * LANES + lane], neg_i, mask=valid)

            _, b1, below1 = locate(kvec)
            krem = kvec - below1

            lax.fori_loop(0, 256, zero_hist, 0, unroll=8)

            def colB(j, _):
                key = keys[pl.ds(j * LANES, LANES)]
                hi = lax.shift_right_logical(key, 8)
                lo = jnp.bitwise_and(key, 255)
                plsc.addupdate_scatter(hist, [lo * LANES + lane], ones_i,
                                       mask=hi == b1)
                return 0
            lax.fori_loop(0, L, colB, 0, unroll=4)

            for key, valid in exk:
                hi = lax.shift_right_logical(key, 8)
                lo = jnp.bitwise_and(key, 255)
                m2 = jnp.logical_and(valid, hi == b1)
                plsc.addupdate_scatter(hist, [lo * LANES + lane], neg_i,
                                       mask=m2)

            _, b2, below2 = locate(krem)

            taub[pl.ds(g * LANES, LANES)] = b1 * 256 + b2
            cntb[pl.ds(g * LANES, LANES)] = below1 + below2
            return 0

        lax.fori_loop(0, NG, do_group, 0)
        pltpu.sync_copy(taub, tau_hbm.at[pl.ds(out0, RPW)])
        pltpu.sync_copy(cntb, cnt_hbm.at[pl.ds(out0, RPW)])

    return sc_kernel(xf, yf, zf)


def _tc_bisect_body(r_ref, rt_ref, o_ref):
    rows = r_ref[0]
    cols = rt_ref[0]
    blk = pl.program_id(1)
    RB = rows.shape[0]
    L = cols.shape[1]

    ni = (rows[:, 0:1] * rows[:, 0:1] + rows[:, 1:2] * rows[:, 1:2]
          + rows[:, 2:3] * rows[:, 2:3])
    nj = (cols[0:1, :] * cols[0:1, :] + cols[1:2, :] * cols[1:2, :]
          + cols[2:3, :] * cols[2:3, :])
    dot = jnp.dot(rows, cols, preferred_element_type=jnp.float32)
    d2 = jnp.maximum(ni + nj - 2.0 * dot, 0.0)

    i_idx = blk * RB + lax.broadcasted_iota(jnp.int32, (RB, 1), 0)
    j_idx = lax.broadcasted_iota(jnp.int32, (1, L), 1)
    excl = jnp.abs(i_idx - j_idx) <= EXCLUDE
    d2m = jnp.where(excl, MASK_D2, d2)

    v = lax.shift_right_logical(lax.bitcast_convert_type(d2m, jnp.int32), 15)

    def bit_step(i, q):
        bit = lax.shift_left(jnp.int32(1), jnp.int32(15) - i)
        cand = q | bit
        cnt = jnp.sum((v < cand).astype(jnp.int32), axis=1, keepdims=True)
        return jnp.where(cnt <= K - 1, cand, q)

    q0 = jnp.zeros((RB, 1), dtype=jnp.int32)
    q = lax.fori_loop(0, 16, bit_step, q0)

    tau2 = lax.bitcast_convert_type(
        lax.shift_left(q, 15) + jnp.int32(0x4000), jnp.float32)
    sel = v < q
    c = jnp.sum(sel.astype(jnp.float32), axis=1, keepdims=True)

    r = jnp.sqrt(d2m + 1e-12)
    f = _pair_energy(r)
    ssum = jnp.sum(jnp.where(sel, f, 0.0), axis=1, keepdims=True)
    f_tau = _pair_energy(jnp.sqrt(tau2 + 1e-12))
    row_total = ssum + (K - c) * f_tau
    o_ref[0, 0] = jnp.full((8, 128), jnp.sum(row_total), jnp.float32)


def _tc_body(row_off, r_ref, rt_ref, tk_ref, c_ref, o_ref):
    rows = r_ref[0]
    cols = rt_ref[0]
    tk = tk_ref[0]
    c = c_ref[0].astype(jnp.float32)
    blk = pl.program_id(1)
    RB = rows.shape[0]
    L = cols.shape[1]

    ni = (rows[:, 0:1] * rows[:, 0:1] + rows[:, 1:2] * rows[:, 1:2]
          + rows[:, 2:3] * rows[:, 2:3])
    nj = (cols[0:1, :] * cols[0:1, :] + cols[1:2, :] * cols[1:2, :]
          + cols[2:3, :] * cols[2:3, :])
    dot = jnp.dot(rows, cols, preferred_element_type=jnp.float32)
    d2 = jnp.maximum(ni + nj - 2.0 * dot, 0.0)

    i_idx = row_off + blk * RB + lax.broadcasted_iota(jnp.int32, (RB, 1), 0)
    j_idx = lax.broadcasted_iota(jnp.int32, (1, L), 1)
    excl = jnp.abs(i_idx - j_idx) <= EXCLUDE
    d2m = jnp.where(excl, MASK_D2, d2)

    v = lax.shift_right_logical(lax.bitcast_convert_type(d2m, jnp.int32), 15)
    sel = v < tk

    r = jnp.sqrt(d2m + 1e-12)
    f = _pair_energy(r)
    s = jnp.sum(jnp.where(sel, f, 0.0), axis=1, keepdims=True)

    tau2 = lax.bitcast_convert_type(
        lax.shift_left(tk, 15) + jnp.int32(0x4000), jnp.float32)
    f_tau = _pair_energy(jnp.sqrt(tau2 + 1e-12))
    row_total = s + (K - c) * f_tau
    o_ref[0, 0] = jnp.full((8, 128), jnp.sum(row_total), jnp.float32)


@jax.jit
def kernel(R):
    B, L, _ = R.shape
    RT = jnp.swapaxes(R, 1, 2)
    nblk_a = SPLIT // _RB
    nblk_b = (L - SPLIT) // _RB
    totals = []
    for b in range(B):
        xf = RT[b, 0, :]
        yf = RT[b, 1, :]
        zf = RT[b, 2, :]
        tau_key, below = _sc_select(xf, yf, zf, L)
        tk3 = tau_key.reshape(1, L - SPLIT, 1)
        c3 = below.reshape(1, L - SPLIT, 1)
        out_a = pl.pallas_call(
            _tc_bisect_body,
            grid=(1, nblk_a),
            in_specs=[
                pl.BlockSpec((1, _RB, 3), lambda i, k: (i, k, 0)),
                pl.BlockSpec((1, 3, L), lambda i, k: (i, 0, 0)),
            ],
            out_specs=pl.BlockSpec((1, 1, 8, 128),
                                   lambda i, k: (i, k, 0, 0)),
            out_shape=jax.ShapeDtypeStruct((1, nblk_a, 8, 128),
                                           jnp.float32),
        )(R[b:b + 1, :SPLIT], RT[b:b + 1])
        out_b = pl.pallas_call(
            functools.partial(_tc_body, SPLIT),
            grid=(1, nblk_b),
            in_specs=[
                pl.BlockSpec((1, _RB, 3), lambda i, k: (i, k, 0)),
                pl.BlockSpec((1, 3, L), lambda i, k: (i, 0, 0)),
                pl.BlockSpec((1, _RB, 1), lambda i, k: (i, k, 0)),
                pl.BlockSpec((1, _RB, 1), lambda i, k: (i, k, 0)),
            ],
            out_specs=pl.BlockSpec((1, 1, 8, 128),
                                   lambda i, k: (i, k, 0, 0)),
            out_shape=jax.ShapeDtypeStruct((1, nblk_b, 8, 128),
                                           jnp.float32),
        )(R[b:b + 1, SPLIT:], RT[b:b + 1], tk3, c3)
        totals.append(jnp.sum(out_a[0, :, 0, 0]) +
                      jnp.sum(out_b[0, :, 0, 0]))
    return jnp.stack(totals)
